# baseline (speedup 1.0000x reference)
import functools

import jax
import jax.numpy as jnp
from jax import lax
from jax.experimental import pallas as pl
from jax.experimental.pallas import tpu as pltpu
from jax.experimental.pallas import tpu_sc as plsc


def _make_gather_kernel(B, V, D):
    info = plsc.get_sparse_core_info()
    nw = info.num_cores * info.num_subcores
    assert B % nw == 0
    b_per_w = B // nw
    ch = 16
    n_ch = b_per_w // ch
    nbuf = 4

    mesh = plsc.VectorSubcoreMesh(core_axis_name="c", subcore_axis_name="s")

    @functools.partial(
        pl.kernel,
        mesh=mesh,
        out_type=jax.ShapeDtypeStruct((B, D), jnp.float32),
        scratch_types=[
            pltpu.VMEM((b_per_w,), jnp.int32),
            pltpu.VMEM((nbuf, ch, D), jnp.float32),
            pltpu.SemaphoreType.DMA((nbuf,)),
            pltpu.SemaphoreType.DMA((nbuf,)),
        ],
    )
    def k(labels_hbm, table_hbm, out_hbm, idx_v, rows_v, gsem, osem):
        wid = lax.axis_index("s") * info.num_cores + lax.axis_index("c")
        base = wid * b_per_w
        pltpu.sync_copy(labels_hbm.at[pl.ds(base, b_per_w)], idx_v)

        def gather_start(j):
            slot = j % nbuf
            return pltpu.async_copy(
                table_hbm.at[idx_v.at[pl.ds(j * ch, ch)]],
                rows_v.at[slot],
                gsem.at[slot],
            )

        def out_start(j):
            slot = j % nbuf
            return pltpu.async_copy(
                rows_v.at[slot],
                out_hbm.at[pl.ds(base + j * ch, ch)],
                osem.at[slot],
            )

        gh = [None] * n_ch
        oh = [None] * n_ch
        gh[0] = gather_start(0)
        for j in range(n_ch):
            if j + 1 < n_ch:
                if j + 1 >= nbuf:
                    oh[j + 1 - nbuf].wait()
                gh[j + 1] = gather_start(j + 1)
            gh[j].wait()
            oh[j] = out_start(j)
        for j in range(max(0, n_ch - nbuf), n_ch):
            oh[j].wait()

    return k


def _make_tc_gather(B, V, D, rows_per_blk=512):
    grid = B // rows_per_blk

    def body(labels_ref, table_ref, out_ref):
        base = pl.program_id(0) * rows_per_blk
        for r in range(rows_per_blk):
            lbl = labels_ref[base + r]
            out_ref[pl.ds(r, 1), :] = table_ref[pl.ds(lbl, 1), :]

    return pl.pallas_call(
        body,
        grid_spec=pltpu.PrefetchScalarGridSpec(
            num_scalar_prefetch=1,
            grid=(grid,),
            in_specs=[pl.BlockSpec((V, D), lambda i, lref: (0, 0))],
            out_specs=pl.BlockSpec((rows_per_blk, D), lambda i, lref: (i, 0)),
        ),
        out_shape=jax.ShapeDtypeStruct((B, D), jnp.float32),
    )


def _make_tc_split_gather(B, V, D, rows_per_blk=512, mxu_rows=256):
    grid = B // rows_per_blk
    cp_rows = rows_per_blk - mxu_rows

    def body(labels_ref, lbl_vec_ref, table_ref, table_bf16_ref, out_ref):
        base = pl.program_id(0) * rows_per_blk
        for r in range(cp_rows):
            lbl = labels_ref[base + r]
            out_ref[pl.ds(r, 1), :] = table_ref[pl.ds(lbl, 1), :]
        lbl = lbl_vec_ref[pl.ds(cp_rows, mxu_rows), :]
        cls = jax.lax.broadcasted_iota(jnp.int32, (mxu_rows, V), 1)
        onehot = (lbl == cls).astype(jnp.bfloat16)
        out_ref[pl.ds(cp_rows, mxu_rows), :] = jnp.dot(
            onehot, table_bf16_ref[...], preferred_element_type=jnp.float32
        )

    return pl.pallas_call(
        body,
        grid_spec=pltpu.PrefetchScalarGridSpec(
            num_scalar_prefetch=1,
            grid=(grid,),
            in_specs=[
                pl.BlockSpec((rows_per_blk, 1), lambda i, lref: (i, 0)),
                pl.BlockSpec((V, D), lambda i, lref: (0, 0)),
                pl.BlockSpec((V, D), lambda i, lref: (0, 0)),
            ],
            out_specs=pl.BlockSpec((rows_per_blk, D), lambda i, lref: (i, 0)),
        ),
        out_shape=jax.ShapeDtypeStruct((B, D), jnp.float32),
    )


def _make_tc_onehot_gather(B, V, D, rows_per_blk=512):
    grid = B // rows_per_blk

    def body(labels_ref, table_ref, out_ref):
        lbl = labels_ref[...]
        cls = jax.lax.broadcasted_iota(jnp.int32, (rows_per_blk, V), 1)
        onehot = (lbl == cls).astype(jnp.bfloat16)
        out_ref[...] = jnp.dot(
            onehot, table_ref[...], preferred_element_type=jnp.float32
        )

    return pl.pallas_call(
        body,
        grid=(grid,),
        in_specs=[
            pl.BlockSpec((rows_per_blk, 1), lambda i: (i, 0)),
            pl.BlockSpec((V, D), lambda i: (0, 0)),
        ],
        out_specs=pl.BlockSpec((rows_per_blk, D), lambda i: (i, 0)),
        out_shape=jax.ShapeDtypeStruct((B, D), jnp.float32),
    )


def kernel(labels, table, train):
    del train
    B = labels.shape[0]
    V, D = table.shape
    k = _make_tc_split_gather(B, V, D)
    labels32 = labels.astype(jnp.int32)
    return k(labels32, labels32.reshape(B, 1), table, table.astype(jnp.bfloat16))


# speedup vs baseline: 1.5328x; 1.5328x over previous
import functools

import jax
import jax.numpy as jnp
from jax import lax
from jax.experimental import pallas as pl
from jax.experimental.pallas import tpu as pltpu
from jax.experimental.pallas import tpu_sc as plsc


def _make_gather_kernel(B, V, D):
    info = plsc.get_sparse_core_info()
    nw = info.num_cores * info.num_subcores
    assert B % nw == 0
    b_per_w = B // nw
    ch = 16
    n_ch = b_per_w // ch
    nbuf = 4

    mesh = plsc.VectorSubcoreMesh(core_axis_name="c", subcore_axis_name="s")

    @functools.partial(
        pl.kernel,
        mesh=mesh,
        out_type=jax.ShapeDtypeStruct((B, D), jnp.float32),
        scratch_types=[
            pltpu.VMEM((b_per_w,), jnp.int32),
            pltpu.VMEM((nbuf, ch, D), jnp.float32),
            pltpu.SemaphoreType.DMA((nbuf,)),
            pltpu.SemaphoreType.DMA((nbuf,)),
        ],
    )
    def k(labels_hbm, table_hbm, out_hbm, idx_v, rows_v, gsem, osem):
        wid = lax.axis_index("s") * info.num_cores + lax.axis_index("c")
        base = wid * b_per_w
        pltpu.sync_copy(labels_hbm.at[pl.ds(base, b_per_w)], idx_v)

        def gather_start(j):
            slot = j % nbuf
            return pltpu.async_copy(
                table_hbm.at[idx_v.at[pl.ds(j * ch, ch)]],
                rows_v.at[slot],
                gsem.at[slot],
            )

        def out_start(j):
            slot = j % nbuf
            return pltpu.async_copy(
                rows_v.at[slot],
                out_hbm.at[pl.ds(base + j * ch, ch)],
                osem.at[slot],
            )

        gh = [None] * n_ch
        oh = [None] * n_ch
        gh[0] = gather_start(0)
        for j in range(n_ch):
            if j + 1 < n_ch:
                if j + 1 >= nbuf:
                    oh[j + 1 - nbuf].wait()
                gh[j + 1] = gather_start(j + 1)
            gh[j].wait()
            oh[j] = out_start(j)
        for j in range(max(0, n_ch - nbuf), n_ch):
            oh[j].wait()

    return k


def _make_tc_gather(B, V, D, rows_per_blk=512):
    grid = B // rows_per_blk

    def body(labels_ref, table_ref, out_ref):
        base = pl.program_id(0) * rows_per_blk
        for r in range(rows_per_blk):
            lbl = labels_ref[base + r]
            out_ref[pl.ds(r, 1), :] = table_ref[pl.ds(lbl, 1), :]

    return pl.pallas_call(
        body,
        grid_spec=pltpu.PrefetchScalarGridSpec(
            num_scalar_prefetch=1,
            grid=(grid,),
            in_specs=[pl.BlockSpec((V, D), lambda i, lref: (0, 0))],
            out_specs=pl.BlockSpec((rows_per_blk, D), lambda i, lref: (i, 0)),
        ),
        out_shape=jax.ShapeDtypeStruct((B, D), jnp.float32),
    )


def _make_tc_split_gather(B, V, D, rows_per_blk=512, mxu_rows=256):
    grid = B // rows_per_blk
    cp_rows = rows_per_blk - mxu_rows

    def body(labels_ref, lbl_vec_ref, table_ref, table_bf16_ref, out_ref):
        base = pl.program_id(0) * rows_per_blk
        for r in range(cp_rows):
            lbl = labels_ref[base + r]
            out_ref[pl.ds(r, 1), :] = table_ref[pl.ds(lbl, 1), :]
        lbl = lbl_vec_ref[pl.ds(cp_rows, mxu_rows), :]
        cls = jax.lax.broadcasted_iota(jnp.int32, (mxu_rows, V), 1)
        onehot = (lbl == cls).astype(jnp.bfloat16)
        out_ref[pl.ds(cp_rows, mxu_rows), :] = jnp.dot(
            onehot, table_bf16_ref[...], preferred_element_type=jnp.float32
        )

    return pl.pallas_call(
        body,
        grid_spec=pltpu.PrefetchScalarGridSpec(
            num_scalar_prefetch=1,
            grid=(grid,),
            in_specs=[
                pl.BlockSpec((rows_per_blk, 1), lambda i, lref: (i, 0)),
                pl.BlockSpec((V, D), lambda i, lref: (0, 0)),
                pl.BlockSpec((V, D), lambda i, lref: (0, 0)),
            ],
            out_specs=pl.BlockSpec((rows_per_blk, D), lambda i, lref: (i, 0)),
        ),
        out_shape=jax.ShapeDtypeStruct((B, D), jnp.float32),
    )


def _make_tc_onehot_gather(B, V, D, rows_per_blk=512):
    grid = B // rows_per_blk

    def body(labels_ref, table_ref, out_ref):
        lbl = labels_ref[...]
        cls = jax.lax.broadcasted_iota(jnp.int32, (rows_per_blk, V), 1)
        onehot = (lbl == cls).astype(jnp.bfloat16)
        out_ref[...] = jnp.dot(
            onehot, table_ref[...], preferred_element_type=jnp.float32
        )

    return pl.pallas_call(
        body,
        grid=(grid,),
        in_specs=[
            pl.BlockSpec((rows_per_blk, 1), lambda i: (i, 0)),
            pl.BlockSpec((V, D), lambda i: (0, 0)),
        ],
        out_specs=pl.BlockSpec((rows_per_blk, D), lambda i: (i, 0)),
        out_shape=jax.ShapeDtypeStruct((B, D), jnp.float32),
    )


def _make_tc_aligned_gather(B, V, D, rows_per_blk=512, d_pad=2048):
    grid = B // rows_per_blk

    def body(labels_ref, table_ref, out_ref, tab_al):
        @pl.when(pl.program_id(0) == 0)
        def _init():
            tab_al[:, :D] = table_ref[...]

        base = pl.program_id(0) * rows_per_blk
        for r in range(rows_per_blk):
            lbl = labels_ref[base + r]
            out_ref[pl.ds(r, 1), :] = tab_al[pl.ds(lbl, 1), :D]

    return pl.pallas_call(
        body,
        grid_spec=pltpu.PrefetchScalarGridSpec(
            num_scalar_prefetch=1,
            grid=(grid,),
            in_specs=[pl.BlockSpec((V, D), lambda i, lref: (0, 0))],
            out_specs=pl.BlockSpec((rows_per_blk, D), lambda i, lref: (i, 0)),
            scratch_shapes=[pltpu.VMEM((V, d_pad), jnp.float32)],
        ),
        out_shape=jax.ShapeDtypeStruct((B, D), jnp.float32),
    )


def kernel(labels, table, train):
    del train
    B = labels.shape[0]
    V, D = table.shape
    k = _make_tc_aligned_gather(B, V, D)
    return k(labels.astype(jnp.int32), table)
